# baseline (speedup 1.0000x reference)
import functools

import jax
import jax.numpy as jnp
from jax import lax
from jax.experimental import pallas as pl
from jax.experimental.pallas import tpu as pltpu
from jax.experimental.pallas import tpu_sc as plsc

N_TOTAL = 4 * 4096 * 2048
NC = 2
NS = 16
NW = NC * NS
PER_W = N_TOTAL // NW
CHUNK = 16384
NCHUNK = PER_W // CHUNK
LANES = 16


def _delta_body(x_hbm, s_hbm, out_hbm, xb, sb, lsem):
    c = lax.axis_index("c")
    s = lax.axis_index("s")
    wid = s * NC + c
    base = wid * PER_W

    def chunk_body(i, carry):
        off = base + i * CHUNK
        cpx = pltpu.async_copy(x_hbm.at[pl.ds(off, CHUNK)], xb, lsem)
        cps = pltpu.async_copy(s_hbm.at[pl.ds(off, CHUNK)], sb, lsem)
        cpx.wait()
        cps.wait()

        def vec_body(j, carry2):
            o = pl.multiple_of(j * LANES, LANES)
            xb[pl.ds(o, LANES)] = xb[pl.ds(o, LANES)] - sb[pl.ds(o, LANES)]
            return carry2

        lax.fori_loop(0, CHUNK // LANES, vec_body, 0, unroll=4)
        pltpu.sync_copy(xb, out_hbm.at[pl.ds(off, CHUNK)])
        return carry

    lax.fori_loop(0, NCHUNK, chunk_body, 0)


@functools.partial(jax.jit, static_argnums=())
def _sc_delta(x_flat, state):
    mesh = plsc.VectorSubcoreMesh(
        core_axis_name="c", subcore_axis_name="s", num_cores=NC, num_subcores=NS
    )
    return pl.kernel(
        _delta_body,
        out_type=jax.ShapeDtypeStruct((N_TOTAL,), jnp.float32),
        mesh=mesh,
        scratch_types=[
            pltpu.VMEM((CHUNK,), jnp.float32),
            pltpu.VMEM((CHUNK,), jnp.float32),
            pltpu.SemaphoreType.DMA,
        ],
    )(x_flat, state)


def kernel(x, state):
    delta_flat = _sc_delta(x.reshape(-1), state)
    return delta_flat.reshape(x.shape)


# speedup vs baseline: 1.2295x; 1.2295x over previous
import functools

import jax
import jax.numpy as jnp
from jax import lax
from jax.experimental import pallas as pl
from jax.experimental.pallas import tpu as pltpu
from jax.experimental.pallas import tpu_sc as plsc

N_TOTAL = 4 * 4096 * 2048
NC = 2
NS = 16
NW = NC * NS
PER_W = N_TOTAL // NW
CHUNK = 16384
NCHUNK = PER_W // CHUNK
NBUF = 2
NGRP = NCHUNK // NBUF
LANES = 16


def _delta_body(x_hbm, s_hbm, out_hbm, xb0, sb0, ob0, xb1, sb1, ob1,
                lsem0, lsem1, ssem0, ssem1):
    c = lax.axis_index("c")
    s = lax.axis_index("s")
    wid = s * NC + c
    base = wid * PER_W
    xb = (xb0, xb1)
    sb = (sb0, sb1)
    ob = (ob0, ob1)
    lsem = (lsem0, lsem1)
    ssem = (ssem0, ssem1)

    def start_load(b, off):
        pltpu.async_copy(x_hbm.at[pl.ds(off, CHUNK)], xb[b], lsem[b])
        pltpu.async_copy(s_hbm.at[pl.ds(off, CHUNK)], sb[b], lsem[b])

    def wait_load(b):
        pltpu.make_async_copy(x_hbm.at[pl.ds(0, CHUNK)], xb[b], lsem[b]).wait()
        pltpu.make_async_copy(s_hbm.at[pl.ds(0, CHUNK)], sb[b], lsem[b]).wait()

    def wait_store(b):
        pltpu.make_async_copy(ob[b], out_hbm.at[pl.ds(0, CHUNK)], ssem[b]).wait()

    for b in range(NBUF):
        start_load(b, base + b * CHUNK)

    def group_body(g, carry):
        for b in range(NBUF):
            off = base + (g * NBUF + b) * CHUNK
            wait_load(b)

            @pl.when(g > 0)
            def _():
                wait_store(b)

            def vec_body(j, carry2):
                o = pl.multiple_of(j * LANES, LANES)
                ob[b][pl.ds(o, LANES)] = (
                    xb[b][pl.ds(o, LANES)] - sb[b][pl.ds(o, LANES)]
                )
                return carry2

            lax.fori_loop(0, CHUNK // LANES, vec_body, 0, unroll=8)
            pltpu.async_copy(ob[b], out_hbm.at[pl.ds(off, CHUNK)], ssem[b])

            @pl.when(g < NGRP - 1)
            def _():
                start_load(b, off + NBUF * CHUNK)

        return carry

    lax.fori_loop(0, NGRP, group_body, 0)
    for b in range(NBUF):
        wait_store(b)


@functools.partial(jax.jit, static_argnums=())
def _sc_delta(x_flat, state):
    mesh = plsc.VectorSubcoreMesh(
        core_axis_name="c", subcore_axis_name="s", num_cores=NC, num_subcores=NS
    )
    return pl.kernel(
        _delta_body,
        out_type=jax.ShapeDtypeStruct((N_TOTAL,), jnp.float32),
        mesh=mesh,
        scratch_types=[
            pltpu.VMEM((CHUNK,), jnp.float32),
            pltpu.VMEM((CHUNK,), jnp.float32),
            pltpu.VMEM((CHUNK,), jnp.float32),
            pltpu.VMEM((CHUNK,), jnp.float32),
            pltpu.VMEM((CHUNK,), jnp.float32),
            pltpu.VMEM((CHUNK,), jnp.float32),
            pltpu.SemaphoreType.DMA,
            pltpu.SemaphoreType.DMA,
            pltpu.SemaphoreType.DMA,
            pltpu.SemaphoreType.DMA,
        ],
    )(x_flat, state)


def kernel(x, state):
    delta_flat = _sc_delta(x.reshape(-1), state)
    return delta_flat.reshape(x.shape)
